# baseline (speedup 1.0000x reference)
import jax
import jax.numpy as jnp
from jax import lax
from jax.experimental import pallas as pl
from jax.experimental.pallas import tpu as pltpu, tpu_sc as plsc

HIDDEN = 1024
N_IDX = 4 * 8192

_info = plsc.get_sparse_core_info()
NC, NS = _info.num_cores, _info.num_subcores
NW = NC * NS
B_PER_W = N_IDX // NW
CHUNK = 32
NBUF = 3
N_CHUNKS = B_PER_W // CHUNK
FULL_ROUNDS = N_CHUNKS // NBUF - 1
TAIL_START = (FULL_ROUNDS + 1) * NBUF


def _gather_body(table_hbm, idx_hbm, out_hbm, idx_v, rows_v,
                 gsem0, gsem1, gsem2, osem0, osem1, osem2):
    gsem = (gsem0, gsem1, gsem2)
    osem = (osem0, osem1, osem2)
    wid = lax.axis_index("s") * NC + lax.axis_index("c")
    base = wid * B_PER_W
    pltpu.sync_copy(idx_hbm.at[pl.ds(base, B_PER_W)], idx_v)

    def gather(g, b):
        return pltpu.make_async_copy(
            table_hbm.at[idx_v.at[pl.ds(g * CHUNK, CHUNK)]],
            rows_v.at[b], gsem[b])

    def put(g, b):
        return pltpu.make_async_copy(
            rows_v.at[b], out_hbm.at[pl.ds(base + g * CHUNK, CHUNK)], osem[b])

    for b in range(NBUF):
        gather(b, b).start()

    def round_body(r, _):
        for b in range(NBUF):
            g = r * NBUF + b
            gather(g, b).wait()
            put(g, b).start()
        for b in range(NBUF):
            g = r * NBUF + b
            put(g, b).wait()
            gather(g + NBUF, b).start()
        return _

    lax.fori_loop(0, FULL_ROUNDS, round_body, None)

    for g in range(FULL_ROUNDS * NBUF, N_CHUNKS):
        b = g % NBUF
        gather(g, b).wait()
        put(g, b).start()
        ng = g + NBUF
        if ng < N_CHUNKS:
            put(g, b).wait()
            gather(ng, b).start()
    for g in range(N_CHUNKS - NBUF, N_CHUNKS):
        put(g, g % NBUF).wait()


_mesh = plsc.VectorSubcoreMesh(core_axis_name="c", subcore_axis_name="s")

_gather = pl.kernel(
    _gather_body,
    mesh=_mesh,
    out_type=jax.ShapeDtypeStruct((N_IDX, HIDDEN), jnp.float32),
    scratch_types=[
        pltpu.VMEM((B_PER_W,), jnp.int32),
        pltpu.VMEM((NBUF, CHUNK, HIDDEN), jnp.float32),
        pltpu.SemaphoreType.DMA,
        pltpu.SemaphoreType.DMA,
        pltpu.SemaphoreType.DMA,
        pltpu.SemaphoreType.DMA,
        pltpu.SemaphoreType.DMA,
        pltpu.SemaphoreType.DMA,
    ],
)


def kernel(pos_enc, position_ids):
    b, s = position_ids.shape
    idx = position_ids.reshape(-1).astype(jnp.int32)
    out = _gather(pos_enc, idx)
    return out.reshape(b, s, pos_enc.shape[1])


# speedup vs baseline: 1.0200x; 1.0200x over previous
import jax
import jax.numpy as jnp
from jax import lax
from jax.experimental import pallas as pl
from jax.experimental.pallas import tpu as pltpu, tpu_sc as plsc

HIDDEN = 1024
N_IDX = 4 * 8192

_info = plsc.get_sparse_core_info()
NC, NS = _info.num_cores, _info.num_subcores
NW = NC * NS
B_PER_W = N_IDX // NW
CHUNK = 16
NBUF = 7
N_CHUNKS = B_PER_W // CHUNK
FULL_ROUNDS = N_CHUNKS // NBUF - 1
TAIL_START = (FULL_ROUNDS + 1) * NBUF


def _gather_body(table_hbm, idx_hbm, out_hbm, idx_v, rows_v, *sems):
    gsem = sems[:NBUF]
    osem = sems[NBUF:]
    wid = lax.axis_index("s") * NC + lax.axis_index("c")
    base = wid * B_PER_W
    pltpu.sync_copy(idx_hbm.at[pl.ds(base, B_PER_W)], idx_v)

    def gather(g, b):
        return pltpu.make_async_copy(
            table_hbm.at[idx_v.at[pl.ds(g * CHUNK, CHUNK)]],
            rows_v.at[b], gsem[b])

    def put(g, b):
        return pltpu.make_async_copy(
            rows_v.at[b], out_hbm.at[pl.ds(base + g * CHUNK, CHUNK)], osem[b])

    for b in range(NBUF):
        gather(b, b).start()

    def round_body(r, _):
        for b in range(NBUF):
            g = r * NBUF + b
            gather(g, b).wait()
            put(g, b).start()
        for b in range(NBUF):
            g = r * NBUF + b
            put(g, b).wait()
            gather(g + NBUF, b).start()
        return _

    lax.fori_loop(0, FULL_ROUNDS, round_body, None)

    for g in range(FULL_ROUNDS * NBUF, N_CHUNKS):
        b = g % NBUF
        gather(g, b).wait()
        put(g, b).start()
        ng = g + NBUF
        if ng < N_CHUNKS:
            put(g, b).wait()
            gather(ng, b).start()
    for g in range(N_CHUNKS - NBUF, N_CHUNKS):
        put(g, g % NBUF).wait()


_mesh = plsc.VectorSubcoreMesh(core_axis_name="c", subcore_axis_name="s")

_gather = pl.kernel(
    _gather_body,
    mesh=_mesh,
    out_type=jax.ShapeDtypeStruct((N_IDX, HIDDEN), jnp.float32),
    scratch_types=[
        pltpu.VMEM((B_PER_W,), jnp.int32),
        pltpu.VMEM((NBUF, CHUNK, HIDDEN), jnp.float32),
    ] + [pltpu.SemaphoreType.DMA] * (2 * NBUF),
)


def kernel(pos_enc, position_ids):
    b, s = position_ids.shape
    idx = position_ids.reshape(-1).astype(jnp.int32)
    out = _gather(pos_enc, idx)
    return out.reshape(b, s, pos_enc.shape[1])


# speedup vs baseline: 1.0611x; 1.0403x over previous
import jax
import jax.numpy as jnp
from jax import lax
from jax.experimental import pallas as pl
from jax.experimental.pallas import tpu as pltpu, tpu_sc as plsc

HIDDEN = 1024
N_IDX = 4 * 8192

_info = plsc.get_sparse_core_info()
NC, NS = _info.num_cores, _info.num_subcores
NW = NC * NS
B_PER_W = N_IDX // NW
CHUNK = 16
NBUF = 4
N_CHUNKS = B_PER_W // CHUNK


def _gather_body(table_hbm, idx_hbm, out_hbm, idx_v, rows_v, gsem, osem):
    wid = lax.axis_index("s") * NC + lax.axis_index("c")
    base = wid * B_PER_W
    pltpu.sync_copy(idx_hbm.at[pl.ds(base, B_PER_W)], idx_v)

    def chunk_body(g, _):
        b = lax.rem(g, NBUF)
        pltpu.make_async_copy(
            table_hbm.at[idx_v.at[pl.ds(g * CHUNK, CHUNK)]],
            rows_v.at[b], gsem).start()
        pltpu.make_async_copy(
            rows_v.at[b],
            out_hbm.at[pl.ds(base + g * CHUNK, CHUNK)], osem).start()
        return _

    lax.fori_loop(0, N_CHUNKS, chunk_body, None)

    def drain_body(g, _):
        pltpu.make_async_copy(
            table_hbm.at[idx_v.at[pl.ds(0, CHUNK)]],
            rows_v.at[0], gsem).wait()
        pltpu.make_async_copy(
            rows_v.at[0], out_hbm.at[pl.ds(base, CHUNK)], osem).wait()
        return _

    lax.fori_loop(0, N_CHUNKS, drain_body, None)


_mesh = plsc.VectorSubcoreMesh(core_axis_name="c", subcore_axis_name="s")

_gather = pl.kernel(
    _gather_body,
    mesh=_mesh,
    out_type=jax.ShapeDtypeStruct((N_IDX, HIDDEN), jnp.float32),
    scratch_types=[
        pltpu.VMEM((B_PER_W,), jnp.int32),
        pltpu.VMEM((NBUF, CHUNK, HIDDEN), jnp.float32),
        pltpu.SemaphoreType.DMA,
        pltpu.SemaphoreType.DMA,
    ],
)


def kernel(pos_enc, position_ids):
    b, s = position_ids.shape
    idx = position_ids.reshape(-1).astype(jnp.int32)
    out = _gather(pos_enc, idx)
    return out.reshape(b, s, pos_enc.shape[1])
